# baseline (speedup 1.0000x reference)
import functools

import jax
import jax.numpy as jnp
from jax.experimental import pallas as pl
from jax.experimental.pallas import tpu as pltpu
from jax.experimental.pallas import tpu_sc as plsc

_NF = 51
_BLK = 8192
_MOM = 0.1
_EPS = 1e-5
_NW = 32


def _tca_kernel(lt_ref, labels_ref, pt_ref, pred_ref, diag_ref):
    lt = lt_ref[...]
    lab = labels_ref[0]
    riota = jax.lax.broadcasted_iota(jnp.int32, (_NF, _BLK), 0)

    ml = jnp.where(riota >= 1, lt, -jnp.inf)
    mx = jnp.max(ml, axis=0, keepdims=True)
    pred = jnp.min(jnp.where(ml == mx, riota, _NF + 1), axis=0,
                   keepdims=True)

    mfull = jnp.maximum(mx, lt[0:1, :])
    e = jnp.exp(lt - mfull)
    probs = e / jnp.sum(e, axis=0, keepdims=True)
    pt_ref[...] = probs

    fg = lab != 0
    oh_lab = ((riota == lab) & fg).astype(jnp.float32)
    pred_ref[0] = pred
    diag_ref[0] = jnp.sum(probs * oh_lab, axis=0, keepdims=True)


def _sc_stats(lab_flat, pred_flat, diag_flat):
    b = lab_flat.shape[0]
    per = b // _NW
    mesh = plsc.VectorSubcoreMesh(core_axis_name="c", subcore_axis_name="s")

    @functools.partial(
        pl.kernel,
        out_type=[
            jax.ShapeDtypeStruct((_NW, _NF, 64), jnp.float32),
            jax.ShapeDtypeStruct((_NW, 2, 64), jnp.float32),
        ],
        mesh=mesh,
        compiler_params=pltpu.CompilerParams(needs_layout_passes=False),
        scratch_types=[
            pltpu.VMEM((per,), jnp.int32),
            pltpu.VMEM((per,), jnp.int32),
            pltpu.VMEM((per,), jnp.float32),
            pltpu.VMEM((16 * (_NF + 2) * 64,), jnp.float32),
            pltpu.VMEM((_NF + 2, 64), jnp.float32),
        ],
    )
    def run(lab_hbm, pred_hbm, diag_hbm, ct_hbm, spcnt_hbm,
            lab_v, pred_v, diag_v, tab, red):
        wid = jax.lax.axis_index("s") * 2 + jax.lax.axis_index("c")
        base = wid * per
        pltpu.sync_copy(lab_hbm.at[pl.ds(base, per)], lab_v)
        pltpu.sync_copy(pred_hbm.at[pl.ds(base, per)], pred_v)
        pltpu.sync_copy(diag_hbm.at[pl.ds(base, per)], diag_v)

        z = jnp.zeros((16,), jnp.float32)
        lstride = (_NF + 2) * 64

        def zero_chunk(j, carry):
            for cc in range(4):
                tab[pl.ds(j * 64 + cc * 16, 16)] = z
            return carry

        jax.lax.fori_loop(0, 16 * (_NF + 2), zero_chunk, 0)

        lanes = jax.lax.broadcasted_iota(jnp.int32, (16,), 0)
        lane_off = lanes * lstride
        ones = jnp.ones((16,), jnp.float32)

        def hist(c, carry):
            lab = lab_v[pl.ds(c * 16, 16)]
            prd = pred_v[pl.ds(c * 16, 16)]
            dg = diag_v[pl.ds(c * 16, 16)]
            msk = lab != 0
            kct = lane_off + prd * 64 + lab
            plsc.addupdate_scatter(tab, [kct], ones, mask=msk)
            ksp = lane_off + (_NF * 64) + lab
            plsc.addupdate_scatter(tab, [ksp], dg, mask=msk)
            plsc.addupdate_scatter(tab, [ksp + 64], ones, mask=msk)
            return carry

        jax.lax.fori_loop(0, per // 16, hist, 0)

        def lane_reduce(r, carry):
            for cc in range(4):
                acc = tab[pl.ds(r * 64 + cc * 16, 16)]
                for lane in range(1, 16):
                    acc = acc + tab[pl.ds(lane * lstride + r * 64 + cc * 16,
                                          16)]
                red[r, pl.ds(cc * 16, 16)] = acc
            return carry

        jax.lax.fori_loop(0, _NF + 2, lane_reduce, 0)

        pltpu.sync_copy(red.at[pl.ds(0, _NF)], ct_hbm.at[wid])
        pltpu.sync_copy(red.at[pl.ds(_NF, 2)], spcnt_hbm.at[wid])

    return run(lab_flat, pred_flat, diag_flat)


def _tcb_kernel(pt_ref, ct_ref, spcnt_ref, rlp_ref, rcpt_ref, out_ref,
                w2_scr):
    i = pl.program_id(0)

    @pl.when(i == 0)
    def _solve():
        ct = jnp.sum(ct_ref[...], axis=0)[:, :_NF]
        sc = jnp.sum(spcnt_ref[...], axis=0)
        sp_col = sc[0:1, :_NF]
        cnt_col = sc[1:2, :_NF]
        m_t = ct + rcpt_ref[...]
        cn = m_t / jnp.sum(m_t, axis=1, keepdims=True)
        lp = sp_col / cnt_col
        rlp = rlp_ref[...]
        rlp_new = jnp.where(cnt_col > 0.0,
                            _MOM * lp + (1.0 - _MOM) * rlp, rlp)
        rp = jax.lax.dot_general(
            rlp_new, cn, (((1,), (0,)), ((), ())),
            preferred_element_type=jnp.float32)
        w2_scr[...] = cn / (rp + _EPS)

    pt = pt_ref[...]
    out_ref[...] = jax.lax.dot_general(
        pt, w2_scr[...], (((0,), (0,)), ((), ())),
        preferred_element_type=jnp.float32)


def kernel(relation_logits, rel_labels, running_labeling_prob,
           running_column_prob, running_label):
    b, nf = relation_logits.shape
    nb = b // _BLK
    lt = relation_logits.T
    labels3 = rel_labels.reshape(nb, 1, _BLK)
    rlp2 = running_labeling_prob.reshape(1, nf)
    rcpt = running_column_prob.T

    probs_t, pred3, diag3 = pl.pallas_call(
        _tca_kernel,
        grid=(nb,),
        in_specs=[
            pl.BlockSpec((nf, _BLK), lambda i: (0, i)),
            pl.BlockSpec((1, 1, _BLK), lambda i: (i, 0, 0)),
        ],
        out_specs=[
            pl.BlockSpec((nf, _BLK), lambda i: (0, i)),
            pl.BlockSpec((1, 1, _BLK), lambda i: (i, 0, 0)),
            pl.BlockSpec((1, 1, _BLK), lambda i: (i, 0, 0)),
        ],
        out_shape=[
            jax.ShapeDtypeStruct((nf, b), jnp.float32),
            jax.ShapeDtypeStruct((nb, 1, _BLK), jnp.int32),
            jax.ShapeDtypeStruct((nb, 1, _BLK), jnp.float32),
        ],
    )(lt, labels3)

    ct_parts, spcnt_parts = _sc_stats(
        rel_labels, pred3.reshape(b), diag3.reshape(b))

    return pl.pallas_call(
        _tcb_kernel,
        grid=(nb,),
        in_specs=[
            pl.BlockSpec((nf, _BLK), lambda i: (0, i)),
            pl.BlockSpec((_NW, nf, 64), lambda i: (0, 0, 0)),
            pl.BlockSpec((_NW, 2, 64), lambda i: (0, 0, 0)),
            pl.BlockSpec((1, nf), lambda i: (0, 0)),
            pl.BlockSpec((nf, nf), lambda i: (0, 0)),
        ],
        out_specs=pl.BlockSpec((_BLK, nf), lambda i: (i, 0)),
        out_shape=jax.ShapeDtypeStruct((b, nf), jnp.float32),
        scratch_shapes=[
            pltpu.VMEM((nf, nf), jnp.float32),
        ],
    )(probs_t, ct_parts, spcnt_parts, rlp2, rcpt)


# speedup vs baseline: 2.4324x; 2.4324x over previous
import jax
import jax.numpy as jnp
from jax.experimental import pallas as pl
from jax.experimental.pallas import tpu as pltpu

_NF = 51
_BLK = 16384
_MOM = 0.1
_EPS = 1e-5


def _fused_kernel(lt_ref, labels_ref, rlp_ref, rcpt_ref, out_ref,
                  pt_scr, ct_scr, rt_scr, w2_scr):
    phase = pl.program_id(0)
    i = pl.program_id(1)

    @pl.when(phase == 0)
    def _pass0():
        @pl.when(i == 0)
        def _init():
            ct_scr[...] = jnp.zeros_like(ct_scr)
            rt_scr[...] = jnp.zeros_like(rt_scr)

        lt = lt_ref[...]
        lab = labels_ref[0]
        riota = jax.lax.broadcasted_iota(jnp.int32, (_NF, _BLK), 0)

        ml = jnp.where(riota >= 1, lt, -jnp.inf)
        mx = jnp.max(ml, axis=0, keepdims=True)
        pred = jnp.min(jnp.where(ml == mx, riota, _NF + 1), axis=0,
                       keepdims=True)

        mfull = jnp.maximum(mx, lt[0:1, :])
        e = jnp.exp(lt - mfull)
        probs = e / jnp.sum(e, axis=0, keepdims=True)
        pt_scr[:, pl.ds(i * _BLK, _BLK)] = probs

        fg = lab != 0
        oh_lab = ((riota == lab) & fg).astype(jnp.float32)
        oh_pred = ((riota == pred) & fg).astype(jnp.float32)
        ct_scr[...] += jax.lax.dot_general(
            oh_pred, oh_lab, (((1,), (1,)), ((), ())),
            preferred_element_type=jnp.float32)
        rt_scr[...] += jax.lax.dot_general(
            probs, oh_lab, (((1,), (1,)), ((), ())),
            preferred_element_type=jnp.float32)

    @pl.when(phase == 1)
    def _pass1():
        @pl.when(i == 0)
        def _solve():
            ct = ct_scr[...]
            m_t = ct + rcpt_ref[...]
            cn = m_t / jnp.sum(m_t, axis=1, keepdims=True)
            cnt_col = jnp.sum(ct, axis=0, keepdims=True)
            r2 = jax.lax.broadcasted_iota(jnp.int32, (_NF, _NF), 0)
            c2 = jax.lax.broadcasted_iota(jnp.int32, (_NF, _NF), 1)
            eye = (r2 == c2).astype(jnp.float32)
            sp_col = jnp.sum(rt_scr[...] * eye, axis=0, keepdims=True)
            lp = sp_col / cnt_col
            rlp = rlp_ref[...]
            rlp_new = jnp.where(cnt_col > 0.0,
                                _MOM * lp + (1.0 - _MOM) * rlp, rlp)
            rp = jax.lax.dot_general(
                rlp_new, cn, (((1,), (0,)), ((), ())),
                preferred_element_type=jnp.float32)
            w2_scr[...] = cn / (rp + _EPS)

        pt = pt_scr[:, pl.ds(i * _BLK, _BLK)]
        out_ref[...] = jax.lax.dot_general(
            pt, w2_scr[...], (((0,), (0,)), ((), ())),
            preferred_element_type=jnp.float32)


def kernel(relation_logits, rel_labels, running_labeling_prob,
           running_column_prob, running_label):
    b, nf = relation_logits.shape
    nb = b // _BLK
    lt = relation_logits.T
    labels3 = rel_labels.reshape(nb, 1, _BLK)
    rlp2 = running_labeling_prob.reshape(1, nf)
    rcpt = running_column_prob.T
    return pl.pallas_call(
        _fused_kernel,
        grid=(2, nb),
        in_specs=[
            pl.BlockSpec((nf, _BLK), lambda p, i: (0, i * (1 - p))),
            pl.BlockSpec((1, 1, _BLK), lambda p, i: (i * (1 - p), 0, 0)),
            pl.BlockSpec((1, nf), lambda p, i: (0, 0)),
            pl.BlockSpec((nf, nf), lambda p, i: (0, 0)),
        ],
        out_specs=pl.BlockSpec((_BLK, nf), lambda p, i: (i * p, 0)),
        out_shape=jax.ShapeDtypeStruct((b, nf), jnp.float32),
        scratch_shapes=[
            pltpu.VMEM((nf, b), jnp.float32),
            pltpu.VMEM((nf, nf), jnp.float32),
            pltpu.VMEM((nf, nf), jnp.float32),
            pltpu.VMEM((nf, nf), jnp.float32),
        ],
    )(lt, labels3, rlp2, rcpt)


# speedup vs baseline: 2.5519x; 1.0491x over previous
import jax
import jax.numpy as jnp
from jax.experimental import pallas as pl
from jax.experimental.pallas import tpu as pltpu

_NF = 51
_BLK = 8192
_MOM = 0.1
_EPS = 1e-5


def _fused_kernel(lt_ref, labels_ref, rlp_ref, rcpt_ref, out_ref,
                  pt_scr, ct_scr, rt_scr, w2_scr):
    phase = pl.program_id(0)
    i = pl.program_id(1)

    @pl.when(phase == 0)
    def _pass0():
        @pl.when(i == 0)
        def _init():
            ct_scr[...] = jnp.zeros_like(ct_scr)
            rt_scr[...] = jnp.zeros_like(rt_scr)

        lt = lt_ref[...]
        lab = labels_ref[0]
        riota = jax.lax.broadcasted_iota(jnp.int32, (_NF, _BLK), 0)

        ml = jnp.where(riota >= 1, lt, -jnp.inf)
        mx = jnp.max(ml, axis=0, keepdims=True)
        pred = jnp.min(jnp.where(ml == mx, riota, _NF + 1), axis=0,
                       keepdims=True)

        mfull = jnp.maximum(mx, lt[0:1, :])
        e = jnp.exp(lt - mfull)
        probs = e / jnp.sum(e, axis=0, keepdims=True)
        pt_scr[:, pl.ds(i * _BLK, _BLK)] = probs

        fg = lab != 0
        oh_lab = ((riota == lab) & fg).astype(jnp.float32)
        oh_pred = ((riota == pred) & fg).astype(jnp.float32)
        ct_scr[...] += jax.lax.dot_general(
            oh_pred, oh_lab, (((1,), (1,)), ((), ())),
            preferred_element_type=jnp.float32)
        rt_scr[...] += jax.lax.dot_general(
            probs, oh_lab, (((1,), (1,)), ((), ())),
            preferred_element_type=jnp.float32)

    @pl.when(phase == 1)
    def _pass1():
        @pl.when(i == 0)
        def _solve():
            ct = ct_scr[...]
            m_t = ct + rcpt_ref[...]
            cn = m_t / jnp.sum(m_t, axis=1, keepdims=True)
            cnt_col = jnp.sum(ct, axis=0, keepdims=True)
            r2 = jax.lax.broadcasted_iota(jnp.int32, (_NF, _NF), 0)
            c2 = jax.lax.broadcasted_iota(jnp.int32, (_NF, _NF), 1)
            eye = (r2 == c2).astype(jnp.float32)
            sp_col = jnp.sum(rt_scr[...] * eye, axis=0, keepdims=True)
            lp = sp_col / cnt_col
            rlp = rlp_ref[...]
            rlp_new = jnp.where(cnt_col > 0.0,
                                _MOM * lp + (1.0 - _MOM) * rlp, rlp)
            rp = jax.lax.dot_general(
                rlp_new, cn, (((1,), (0,)), ((), ())),
                preferred_element_type=jnp.float32)
            w2_scr[...] = cn / (rp + _EPS)

        pt = pt_scr[:, pl.ds(i * _BLK, _BLK)]
        out_ref[...] = jax.lax.dot_general(
            pt, w2_scr[...], (((0,), (0,)), ((), ())),
            preferred_element_type=jnp.float32)


def kernel(relation_logits, rel_labels, running_labeling_prob,
           running_column_prob, running_label):
    b, nf = relation_logits.shape
    nb = b // _BLK
    lt = relation_logits.T
    labels3 = rel_labels.reshape(nb, 1, _BLK)
    rlp2 = running_labeling_prob.reshape(1, nf)
    rcpt = running_column_prob.T
    return pl.pallas_call(
        _fused_kernel,
        grid=(2, nb),
        in_specs=[
            pl.BlockSpec((nf, _BLK), lambda p, i: (0, i * (1 - p))),
            pl.BlockSpec((1, 1, _BLK), lambda p, i: (i * (1 - p), 0, 0)),
            pl.BlockSpec((1, nf), lambda p, i: (0, 0)),
            pl.BlockSpec((nf, nf), lambda p, i: (0, 0)),
        ],
        out_specs=pl.BlockSpec((_BLK, nf), lambda p, i: (i * p, 0)),
        out_shape=jax.ShapeDtypeStruct((b, nf), jnp.float32),
        scratch_shapes=[
            pltpu.VMEM((nf, b), jnp.float32),
            pltpu.VMEM((nf, nf), jnp.float32),
            pltpu.VMEM((nf, nf), jnp.float32),
            pltpu.VMEM((nf, nf), jnp.float32),
        ],
    )(lt, labels3, rlp2, rcpt)
